# baseline (speedup 1.0000x reference)
import functools

import jax
import jax.numpy as jnp
from jax import lax
from jax.experimental import pallas as pl
from jax.experimental.pallas import tpu as pltpu
from jax.experimental.pallas import tpu_sc as plsc

N = 2048
D = 768
G = 16
EPG = 4
E = G * EPG
NP = 2 * N
B = 128
NB = 96
LP = NB * B
NC, NS = 2, 16
NW = NC * NS
ALPHA = 0.01


def _routing_body(x_ref, mw_ref, mb_ref, rw_ref, rb_ref,
                  dst0_ref, dst1_ref, w0_ref, w1_ref, be_ref, bv_ref,
                  nbu_ref, new_ref, ord_ref, uniq_ref, nue_ref, aux_ref):
    x = x_ref[...]
    ml = jnp.dot(x, mw_ref[...], preferred_element_type=jnp.float32) + mb_ref[...]
    rl = jnp.dot(x, rw_ref[...], preferred_element_type=jnp.float32) + rb_ref[...]

    gi = lax.broadcasted_iota(jnp.int32, (N, G), 1)
    mv = jnp.max(ml, axis=1, keepdims=True)
    tg = jnp.min(jnp.where(ml == mv, gi, G), axis=1, keepdims=True)

    me = jnp.exp(ml - mv)
    mp = me / jnp.sum(me, axis=1, keepdims=True)
    f = jnp.sum(mp, axis=0, keepdims=True) * (1.0 / N)
    aux = ALPHA * G * jnp.sum(f * f, axis=1, keepdims=True)

    gm = (tg == gi).astype(jnp.float32)
    counts = jnp.sum(gm, axis=0, keepdims=True)

    rlj = [rl[:, G * j:G * (j + 1)] for j in range(EPG)]
    mj = jnp.maximum(jnp.maximum(rlj[0], rlj[1]), jnp.maximum(rlj[2], rlj[3]))
    ej = [jnp.exp(t - mj) for t in rlj]
    se = ej[0] + ej[1] + ej[2] + ej[3]
    lbsum = jnp.zeros((1, G), jnp.float32)
    glj = []
    for j in range(EPG):
        pj = ej[j] / se
        fgj = jnp.sum(pj * gm, axis=0, keepdims=True) / jnp.maximum(counts, 1.0)
        lbsum = lbsum + fgj * fgj
        glj.append(jnp.sum(rlj[j] * gm, axis=1, keepdims=True))
    lb = ALPHA * EPG * lbsum
    aux = aux + jnp.sum(jnp.where(counts > 0.0, lb, 0.0), axis=1, keepdims=True)

    v0 = jnp.maximum(jnp.maximum(glj[0], glj[1]), jnp.maximum(glj[2], glj[3]))
    e0 = jnp.where(glj[0] == v0, 0,
         jnp.where(glj[1] == v0, 1,
         jnp.where(glj[2] == v0, 2, 3))).astype(jnp.int32)
    neg = jnp.float32(-1e30)
    gmj = [jnp.where(e0 == j, neg, glj[j]) for j in range(EPG)]
    v1 = jnp.maximum(jnp.maximum(gmj[0], gmj[1]), jnp.maximum(gmj[2], gmj[3]))
    e1 = jnp.where(gmj[0] == v1, 0,
         jnp.where(gmj[1] == v1, 1,
         jnp.where(gmj[2] == v1, 2, 3))).astype(jnp.int32)
    t = jnp.exp(v1 - v0)
    w0 = 1.0 / (1.0 + t)
    w1 = t / (1.0 + t)
    w0_ref[...] = w0
    w1_ref[...] = w1
    aux_ref[...] = aux

    hi = jax.lax.Precision.HIGHEST
    eid0 = tg * EPG + e0
    eid1 = tg * EPG + e1
    eI = lax.broadcasted_iota(jnp.int32, (N, E), 1)
    oh0 = (eid0 == eI).astype(jnp.float32)
    oh1 = (eid1 == eI).astype(jnp.float32)
    ohs = oh0 + oh1
    countf = jnp.sum(ohs, axis=0, keepdims=True)
    pcf = (((countf.astype(jnp.int32) + B - 1) // B) * B).astype(jnp.float32)
    r64 = lax.broadcasted_iota(jnp.int32, (E, E), 0)
    c64 = lax.broadcasted_iota(jnp.int32, (E, E), 1)
    tl = (r64 < c64).astype(jnp.float32)
    pstart = jnp.dot(pcf, tl, preferred_element_type=jnp.float32, precision=hi)

    cb = 128
    rcb = lax.broadcasted_iota(jnp.int32, (cb, cb), 0)
    ccb = lax.broadcasted_iota(jnp.int32, (cb, cb), 1)
    lex = (rcb > ccb).astype(jnp.float32)
    runtot = jnp.zeros((1, E), jnp.float32)
    d0s, d1s = [], []
    for k in range(N // cb):
        sl = slice(cb * k, cb * (k + 1))
        ex = jnp.dot(lex, ohs[sl], preferred_element_type=jnp.float32,
                     precision=hi) + runtot + pstart
        d0s.append(jnp.sum(ex * oh0[sl], axis=1, keepdims=True))
        d1s.append(jnp.sum(ex * oh1[sl], axis=1, keepdims=True))
        runtot = runtot + jnp.sum(ohs[sl], axis=0, keepdims=True)
    dst0_ref[...] = jnp.concatenate(d0s, axis=0).astype(jnp.int32)
    dst1_ref[...] = jnp.concatenate(d1s, axis=0).astype(jnp.int32)

    bs = (lax.broadcasted_iota(jnp.int32, (NB, 1), 0) * B).astype(jnp.float32)
    cmp = (pstart <= bs).astype(jnp.float32)
    be = jnp.clip(jnp.sum(cmp, axis=1, keepdims=True) - 1.0, 0.0, E - 1.0)
    total = jnp.sum(pcf, axis=1, keepdims=True)
    used = (bs < total).astype(jnp.float32)
    lastm = (bs + B >= total).astype(jnp.float32) * used
    lastbe = jnp.sum(be * lastm, axis=0, keepdims=True)
    bef = be * used + lastbe * (1.0 - used)
    be_ref[...] = bef.astype(jnp.int32)
    bv_ref[...] = used.astype(jnp.int32)
    nbu_ref[...] = (total * (1.0 / B)).astype(jnp.int32)

    tu = (r64 <= c64).astype(jnp.float32)
    usede = (pcf > 0.0).astype(jnp.float32)
    orde = jnp.dot(usede, tu, preferred_element_type=jnp.float32,
                   precision=hi) - 1.0
    nue_ref[...] = jnp.sum(usede, axis=1, keepdims=True).astype(jnp.int32)
    jrow = lax.broadcasted_iota(jnp.int32, (NB, E), 0).astype(jnp.float32)
    em = (orde == jrow).astype(jnp.float32) * usede
    eidxf = lax.broadcasted_iota(jnp.int32, (NB, E), 1).astype(jnp.float32)
    uniq_ref[...] = jnp.sum(em * eidxf, axis=1, keepdims=True).astype(jnp.int32)
    bem = (bef == eidxf).astype(jnp.float32)
    ord_ref[...] = jnp.sum(bem * orde, axis=1, keepdims=True).astype(jnp.int32)
    firstb = jnp.sum(bem * pstart, axis=1, keepdims=True)
    new_ref[...] = (used * (bs == firstb).astype(jnp.float32)).astype(jnp.int32)


_routing_call = pl.pallas_call(
    _routing_body,
    out_shape=(
        jax.ShapeDtypeStruct((N, 1), jnp.int32),
        jax.ShapeDtypeStruct((N, 1), jnp.int32),
        jax.ShapeDtypeStruct((N, 1), jnp.float32),
        jax.ShapeDtypeStruct((N, 1), jnp.float32),
        jax.ShapeDtypeStruct((NB, 1), jnp.int32),
        jax.ShapeDtypeStruct((NB, 1), jnp.int32),
        jax.ShapeDtypeStruct((1, 1), jnp.int32),
        jax.ShapeDtypeStruct((NB, 1), jnp.int32),
        jax.ShapeDtypeStruct((NB, 1), jnp.int32),
        jax.ShapeDtypeStruct((NB, 1), jnp.int32),
        jax.ShapeDtypeStruct((1, 1), jnp.int32),
        jax.ShapeDtypeStruct((1, 1), jnp.float32),
    ),
)


def _dispatch_body(x_hbm, dst0_hbm, dst1_hbm, out_hbm, d0_v, d1_v, rows_v, sem):
    nt = N // NW
    wid = lax.axis_index("s") * NC + lax.axis_index("c")
    base = wid * nt
    pltpu.sync_copy(dst0_hbm.at[pl.ds(base, nt)], d0_v)
    pltpu.sync_copy(dst1_hbm.at[pl.ds(base, nt)], d1_v)
    pltpu.sync_copy(x_hbm.at[pl.ds(base, nt)], rows_v)
    c0 = pltpu.async_copy(rows_v, out_hbm.at[d0_v], sem)
    c1 = pltpu.async_copy(rows_v, out_hbm.at[d1_v], sem)
    c0.wait()
    c1.wait()


def _make_dispatch_sc():
    return pl.kernel(
        _dispatch_body,
        out_type=jax.ShapeDtypeStruct((LP, D), jnp.float32),
        mesh=plsc.VectorSubcoreMesh(core_axis_name="c", subcore_axis_name="s",
                                    num_cores=NC, num_subcores=NS),
        scratch_types=[
            pltpu.VMEM((N // NW,), jnp.int32),
            pltpu.VMEM((N // NW,), jnp.int32),
            pltpu.VMEM((N // NW, D), jnp.float32),
            pltpu.SemaphoreType.DMA,
        ],
    )


RING = 4


def _mlp_body(be_ref, bv_ref, nbu_ref, new_ref, ord_ref, uniq_ref, nue_ref,
              xs_ref, w1_hbm, b1_ref, w2_hbm, b2_ref, w3_hbm, b3_ref, out_ref,
              w1b, w2b, w3b, sems):
    b = pl.program_id(0)
    nue = nue_ref[0]

    def fetch(j, r):
        e = uniq_ref[j]
        pltpu.make_async_copy(w1_hbm.at[e], w1b.at[r], sems.at[r]).start()
        pltpu.make_async_copy(w2_hbm.at[e], w2b.at[r], sems.at[r]).start()
        pltpu.make_async_copy(w3_hbm.at[e], w3b.at[r], sems.at[r]).start()

    @pl.when(b == 0)
    def _():
        for j in range(RING - 1):
            @pl.when(j < nue)
            def _():
                fetch(j, j)

    o = ord_ref[b]

    @pl.when(new_ref[b] == 1)
    def _():
        nxt = o + (RING - 1)

        @pl.when(nxt < nue)
        def _():
            fetch(nxt, lax.rem(nxt, RING))

        r = lax.rem(o, RING)
        e = uniq_ref[o]
        pltpu.make_async_copy(w1_hbm.at[e], w1b.at[r], sems.at[r]).wait()
        pltpu.make_async_copy(w2_hbm.at[e], w2b.at[r], sems.at[r]).wait()
        pltpu.make_async_copy(w3_hbm.at[e], w3b.at[r], sems.at[r]).wait()

    @pl.when(bv_ref[b] == 1)
    def _():
        r = lax.rem(o, RING)
        xb = xs_ref[...]
        h = jnp.dot(xb, w1b[r], preferred_element_type=jnp.float32) + b1_ref[0]
        h = jnp.maximum(h, 0.0)
        h = jnp.dot(h, w2b[r], preferred_element_type=jnp.float32) + b2_ref[0] + xb
        h = jnp.maximum(h, 0.0)
        out_ref[...] = jnp.dot(h, w3b[r], preferred_element_type=jnp.float32) + b3_ref[0]


def _expert_spec(shape):
    return pl.BlockSpec(shape, lambda b, be, bv, nbu, new, ordr, uniq, nue: (be[b], 0, 0))


def _row_spec():
    return pl.BlockSpec(
        (B, D),
        lambda b, be, bv, nbu, new, ordr, uniq, nue: (jnp.minimum(b, nbu[0] - 1), 0))


_mlp_call = pl.pallas_call(
    _mlp_body,
    grid_spec=pltpu.PrefetchScalarGridSpec(
        num_scalar_prefetch=7,
        grid=(NB,),
        in_specs=[
            _row_spec(),
            pl.BlockSpec(memory_space=pl.ANY),
            _expert_spec((1, 1, D)),
            pl.BlockSpec(memory_space=pl.ANY),
            _expert_spec((1, 1, D)),
            pl.BlockSpec(memory_space=pl.ANY),
            _expert_spec((1, 1, D)),
        ],
        out_specs=_row_spec(),
        scratch_shapes=[
            pltpu.VMEM((RING, D, D), jnp.float32),
            pltpu.VMEM((RING, D, D), jnp.float32),
            pltpu.VMEM((RING, D, D), jnp.float32),
            pltpu.SemaphoreType.DMA((RING,)),
        ],
    ),
    out_shape=jax.ShapeDtypeStruct((LP, D), jnp.float32),
)


def _combine_body(ys_hbm, p0_hbm, p1_hbm, a_hbm, b_hbm,
                  i0_v, i1_v, a_v, b_v, sem):
    nt = N // NW
    wid = lax.axis_index("s") * NC + lax.axis_index("c")
    base = wid * nt
    pltpu.sync_copy(p0_hbm.at[pl.ds(base, nt)], i0_v)
    pltpu.sync_copy(p1_hbm.at[pl.ds(base, nt)], i1_v)
    pltpu.async_copy(ys_hbm.at[i0_v], a_v, sem).wait()
    pltpu.async_copy(ys_hbm.at[i1_v], b_v, sem).wait()
    pltpu.sync_copy(a_v, a_hbm.at[pl.ds(base, nt)])
    pltpu.sync_copy(b_v, b_hbm.at[pl.ds(base, nt)])


def _make_combine_sc():
    return pl.kernel(
        _combine_body,
        out_type=(jax.ShapeDtypeStruct((N, D), jnp.float32),
                  jax.ShapeDtypeStruct((N, D), jnp.float32)),
        mesh=plsc.VectorSubcoreMesh(core_axis_name="c", subcore_axis_name="s",
                                    num_cores=NC, num_subcores=NS),
        scratch_types=[
            pltpu.VMEM((N // NW,), jnp.int32),
            pltpu.VMEM((N // NW,), jnp.int32),
            pltpu.VMEM((N // NW, D), jnp.float32),
            pltpu.VMEM((N // NW, D), jnp.float32),
            pltpu.SemaphoreType.DMA,
        ],
    )


def _wadd_body(a_ref, b_ref, w0_ref, w1_ref, out_ref):
    out_ref[...] = a_ref[...] * w0_ref[...] + b_ref[...] * w1_ref[...]


_WB = 256
_wadd_call = pl.pallas_call(
    _wadd_body,
    grid=(N // _WB,),
    in_specs=[
        pl.BlockSpec((_WB, D), lambda i: (i, 0)),
        pl.BlockSpec((_WB, D), lambda i: (i, 0)),
        pl.BlockSpec((_WB, 1), lambda i: (i, 0)),
        pl.BlockSpec((_WB, 1), lambda i: (i, 0)),
    ],
    out_specs=pl.BlockSpec((_WB, D), lambda i: (i, 0)),
    out_shape=jax.ShapeDtypeStruct((N, D), jnp.float32),
)


def kernel(x, meta_W, meta_b, router_W, router_b,
           fc1_W, fc1_b, fc2_W, fc2_b, fc3_W, fc3_b):
    rwf = jnp.transpose(router_W, (2, 0, 1)).reshape(E, D).T
    rbf = jnp.transpose(router_b, (1, 0)).reshape(1, E)
    mb = meta_b.reshape(1, G)

    (dst0, dst1, w0, w1, be, bvalid, nbu, newb, ordb, uniq, nue,
     aux) = _routing_call(x, meta_W, mb, rwf, rbf)

    xs = _make_dispatch_sc()(x, dst0.reshape(N), dst1.reshape(N))

    ys = _mlp_call(be.reshape(NB), bvalid.reshape(NB), nbu.reshape(1),
                   newb.reshape(NB), ordb.reshape(NB), uniq.reshape(NB),
                   nue.reshape(1), xs,
                   fc1_W.reshape(E, D, D), fc1_b.reshape(E, 1, D),
                   fc2_W.reshape(E, D, D), fc2_b.reshape(E, 1, D),
                   fc3_W.reshape(E, D, D), fc3_b.reshape(E, 1, D))

    a, b = _make_combine_sc()(ys, dst0.reshape(N), dst1.reshape(N))
    out = _wadd_call(a, b, w0, w1)
    return out, aux.reshape(())


# speedup vs baseline: 1.0072x; 1.0072x over previous
import functools

import jax
import jax.numpy as jnp
from jax import lax
from jax.experimental import pallas as pl
from jax.experimental.pallas import tpu as pltpu
from jax.experimental.pallas import tpu_sc as plsc

N = 2048
D = 768
G = 16
EPG = 4
E = G * EPG
NP = 2 * N
B = 128
NB = 96
LP = NB * B
NC, NS = 2, 16
NW = NC * NS
ALPHA = 0.01


def _routing_body(x_ref, mw_ref, mb_ref, rw_ref, rb_ref,
                  dst0_ref, dst1_ref, w0_ref, w1_ref, be_ref, bv_ref,
                  nbu_ref, aux_ref):
    x = x_ref[...]
    ml = jnp.dot(x, mw_ref[...], preferred_element_type=jnp.float32) + mb_ref[...]
    rl = jnp.dot(x, rw_ref[...], preferred_element_type=jnp.float32) + rb_ref[...]

    gi = lax.broadcasted_iota(jnp.int32, (N, G), 1)
    mv = jnp.max(ml, axis=1, keepdims=True)
    tg = jnp.min(jnp.where(ml == mv, gi, G), axis=1, keepdims=True)

    me = jnp.exp(ml - mv)
    mp = me / jnp.sum(me, axis=1, keepdims=True)
    f = jnp.sum(mp, axis=0, keepdims=True) * (1.0 / N)
    aux = ALPHA * G * jnp.sum(f * f, axis=1, keepdims=True)

    gm = (tg == gi).astype(jnp.float32)
    counts = jnp.sum(gm, axis=0, keepdims=True)

    rlj = [rl[:, G * j:G * (j + 1)] for j in range(EPG)]
    mj = jnp.maximum(jnp.maximum(rlj[0], rlj[1]), jnp.maximum(rlj[2], rlj[3]))
    ej = [jnp.exp(t - mj) for t in rlj]
    se = ej[0] + ej[1] + ej[2] + ej[3]
    lbsum = jnp.zeros((1, G), jnp.float32)
    glj = []
    for j in range(EPG):
        pj = ej[j] / se
        fgj = jnp.sum(pj * gm, axis=0, keepdims=True) / jnp.maximum(counts, 1.0)
        lbsum = lbsum + fgj * fgj
        glj.append(jnp.sum(rlj[j] * gm, axis=1, keepdims=True))
    lb = ALPHA * EPG * lbsum
    aux = aux + jnp.sum(jnp.where(counts > 0.0, lb, 0.0), axis=1, keepdims=True)

    v0 = jnp.maximum(jnp.maximum(glj[0], glj[1]), jnp.maximum(glj[2], glj[3]))
    e0 = jnp.where(glj[0] == v0, 0,
         jnp.where(glj[1] == v0, 1,
         jnp.where(glj[2] == v0, 2, 3))).astype(jnp.int32)
    neg = jnp.float32(-1e30)
    gmj = [jnp.where(e0 == j, neg, glj[j]) for j in range(EPG)]
    v1 = jnp.maximum(jnp.maximum(gmj[0], gmj[1]), jnp.maximum(gmj[2], gmj[3]))
    e1 = jnp.where(gmj[0] == v1, 0,
         jnp.where(gmj[1] == v1, 1,
         jnp.where(gmj[2] == v1, 2, 3))).astype(jnp.int32)
    t = jnp.exp(v1 - v0)
    w0 = 1.0 / (1.0 + t)
    w1 = t / (1.0 + t)
    w0_ref[...] = w0
    w1_ref[...] = w1
    aux_ref[...] = aux

    hi = jax.lax.Precision.HIGHEST
    eid0 = tg * EPG + e0
    eid1 = tg * EPG + e1
    eI = lax.broadcasted_iota(jnp.int32, (N, E), 1)
    oh0 = (eid0 == eI).astype(jnp.float32)
    oh1 = (eid1 == eI).astype(jnp.float32)
    ohs = oh0 + oh1
    countf = jnp.sum(ohs, axis=0, keepdims=True)
    pcf = (((countf.astype(jnp.int32) + B - 1) // B) * B).astype(jnp.float32)
    r64 = lax.broadcasted_iota(jnp.int32, (E, E), 0)
    c64 = lax.broadcasted_iota(jnp.int32, (E, E), 1)
    tl = (r64 < c64).astype(jnp.float32)
    pstart = jnp.dot(pcf, tl, preferred_element_type=jnp.float32, precision=hi)

    cb = 128
    rcb = lax.broadcasted_iota(jnp.int32, (cb, cb), 0)
    ccb = lax.broadcasted_iota(jnp.int32, (cb, cb), 1)
    lex = (rcb > ccb).astype(jnp.float32)
    runtot = jnp.zeros((1, E), jnp.float32)
    d0s, d1s = [], []
    for k in range(N // cb):
        sl = slice(cb * k, cb * (k + 1))
        ex = jnp.dot(lex, ohs[sl], preferred_element_type=jnp.float32,
                     precision=hi) + runtot + pstart
        d0s.append(jnp.sum(ex * oh0[sl], axis=1, keepdims=True))
        d1s.append(jnp.sum(ex * oh1[sl], axis=1, keepdims=True))
        runtot = runtot + jnp.sum(ohs[sl], axis=0, keepdims=True)
    dst0_ref[...] = jnp.concatenate(d0s, axis=0).astype(jnp.int32)
    dst1_ref[...] = jnp.concatenate(d1s, axis=0).astype(jnp.int32)

    bs = (lax.broadcasted_iota(jnp.int32, (NB, 1), 0) * B).astype(jnp.float32)
    cmp = (pstart <= bs).astype(jnp.float32)
    be = jnp.clip(jnp.sum(cmp, axis=1, keepdims=True) - 1.0, 0.0, E - 1.0)
    total = jnp.sum(pcf, axis=1, keepdims=True)
    used = (bs < total).astype(jnp.float32)
    lastm = (bs + B >= total).astype(jnp.float32) * used
    lastbe = jnp.sum(be * lastm, axis=0, keepdims=True)
    be_ref[...] = (be * used + lastbe * (1.0 - used)).astype(jnp.int32)
    bv_ref[...] = used.astype(jnp.int32)
    nbu_ref[...] = (total * (1.0 / B)).astype(jnp.int32)


_routing_call = pl.pallas_call(
    _routing_body,
    out_shape=(
        jax.ShapeDtypeStruct((N, 1), jnp.int32),
        jax.ShapeDtypeStruct((N, 1), jnp.int32),
        jax.ShapeDtypeStruct((N, 1), jnp.float32),
        jax.ShapeDtypeStruct((N, 1), jnp.float32),
        jax.ShapeDtypeStruct((NB, 1), jnp.int32),
        jax.ShapeDtypeStruct((NB, 1), jnp.int32),
        jax.ShapeDtypeStruct((1, 1), jnp.int32),
        jax.ShapeDtypeStruct((1, 1), jnp.float32),
    ),
)


def _dispatch_body(x_hbm, dst0_hbm, dst1_hbm, out_hbm, d0_v, d1_v, rows_v, sem):
    nt = N // NW
    wid = lax.axis_index("s") * NC + lax.axis_index("c")
    base = wid * nt
    pltpu.sync_copy(dst0_hbm.at[pl.ds(base, nt)], d0_v)
    pltpu.sync_copy(dst1_hbm.at[pl.ds(base, nt)], d1_v)
    pltpu.sync_copy(x_hbm.at[pl.ds(base, nt)], rows_v)
    c0 = pltpu.async_copy(rows_v, out_hbm.at[d0_v], sem)
    c1 = pltpu.async_copy(rows_v, out_hbm.at[d1_v], sem)
    c0.wait()
    c1.wait()


def _make_dispatch_sc():
    return pl.kernel(
        _dispatch_body,
        out_type=jax.ShapeDtypeStruct((LP, D), jnp.float32),
        mesh=plsc.VectorSubcoreMesh(core_axis_name="c", subcore_axis_name="s",
                                    num_cores=NC, num_subcores=NS),
        scratch_types=[
            pltpu.VMEM((N // NW,), jnp.int32),
            pltpu.VMEM((N // NW,), jnp.int32),
            pltpu.VMEM((N // NW, D), jnp.float32),
            pltpu.SemaphoreType.DMA,
        ],
    )


def _mlp_body(be_ref, bv_ref, nbu_ref, xs_ref, w1_ref, b1_ref, w2_ref, b2_ref,
              w3_ref, b3_ref, out_ref):
    @pl.when(bv_ref[pl.program_id(0)] == 1)
    def _():
        xb = xs_ref[...]
        h = jnp.dot(xb, w1_ref[0], preferred_element_type=jnp.float32) + b1_ref[0]
        h = jnp.maximum(h, 0.0)
        h = jnp.dot(h, w2_ref[0], preferred_element_type=jnp.float32) + b2_ref[0] + xb
        h = jnp.maximum(h, 0.0)
        out_ref[...] = jnp.dot(h, w3_ref[0], preferred_element_type=jnp.float32) + b3_ref[0]


def _expert_spec(shape):
    return pl.BlockSpec(shape, lambda b, be, bv, nbu: (be[b], 0, 0))


def _row_spec():
    return pl.BlockSpec((B, D), lambda b, be, bv, nbu: (jnp.minimum(b, nbu[0] - 1), 0))


_mlp_call = pl.pallas_call(
    _mlp_body,
    grid_spec=pltpu.PrefetchScalarGridSpec(
        num_scalar_prefetch=3,
        grid=(NB,),
        in_specs=[
            _row_spec(),
            _expert_spec((1, D, D)),
            _expert_spec((1, 1, D)),
            _expert_spec((1, D, D)),
            _expert_spec((1, 1, D)),
            _expert_spec((1, D, D)),
            _expert_spec((1, 1, D)),
        ],
        out_specs=_row_spec(),
    ),
    out_shape=jax.ShapeDtypeStruct((LP, D), jnp.float32),
)


def _combine_body(ys_hbm, p0_hbm, p1_hbm, a_hbm, b_hbm,
                  i0_v, i1_v, a_v, b_v, sem):
    nt = N // NW
    wid = lax.axis_index("s") * NC + lax.axis_index("c")
    base = wid * nt
    pltpu.sync_copy(p0_hbm.at[pl.ds(base, nt)], i0_v)
    pltpu.sync_copy(p1_hbm.at[pl.ds(base, nt)], i1_v)
    pltpu.async_copy(ys_hbm.at[i0_v], a_v, sem).wait()
    pltpu.async_copy(ys_hbm.at[i1_v], b_v, sem).wait()
    pltpu.sync_copy(a_v, a_hbm.at[pl.ds(base, nt)])
    pltpu.sync_copy(b_v, b_hbm.at[pl.ds(base, nt)])


def _make_combine_sc():
    return pl.kernel(
        _combine_body,
        out_type=(jax.ShapeDtypeStruct((N, D), jnp.float32),
                  jax.ShapeDtypeStruct((N, D), jnp.float32)),
        mesh=plsc.VectorSubcoreMesh(core_axis_name="c", subcore_axis_name="s",
                                    num_cores=NC, num_subcores=NS),
        scratch_types=[
            pltpu.VMEM((N // NW,), jnp.int32),
            pltpu.VMEM((N // NW,), jnp.int32),
            pltpu.VMEM((N // NW, D), jnp.float32),
            pltpu.VMEM((N // NW, D), jnp.float32),
            pltpu.SemaphoreType.DMA,
        ],
    )


def _wadd_body(a_ref, b_ref, w0_ref, w1_ref, out_ref):
    out_ref[...] = a_ref[...] * w0_ref[...] + b_ref[...] * w1_ref[...]


_WB = 256
_wadd_call = pl.pallas_call(
    _wadd_body,
    grid=(N // _WB,),
    in_specs=[
        pl.BlockSpec((_WB, D), lambda i: (i, 0)),
        pl.BlockSpec((_WB, D), lambda i: (i, 0)),
        pl.BlockSpec((_WB, 1), lambda i: (i, 0)),
        pl.BlockSpec((_WB, 1), lambda i: (i, 0)),
    ],
    out_specs=pl.BlockSpec((_WB, D), lambda i: (i, 0)),
    out_shape=jax.ShapeDtypeStruct((N, D), jnp.float32),
)


def kernel(x, meta_W, meta_b, router_W, router_b,
           fc1_W, fc1_b, fc2_W, fc2_b, fc3_W, fc3_b):
    rwf = jnp.transpose(router_W, (2, 0, 1)).reshape(E, D).T
    rbf = jnp.transpose(router_b, (1, 0)).reshape(1, E)
    mb = meta_b.reshape(1, G)

    dst0, dst1, w0, w1, be, bvalid, nbu, aux = _routing_call(x, meta_W, mb, rwf, rbf)

    xs = _make_dispatch_sc()(x, dst0.reshape(N), dst1.reshape(N))

    ys = _mlp_call(be.reshape(NB), bvalid.reshape(NB), nbu.reshape(1), xs,
                   fc1_W.reshape(E, D, D), fc1_b.reshape(E, 1, D),
                   fc2_W.reshape(E, D, D), fc2_b.reshape(E, 1, D),
                   fc3_W.reshape(E, D, D), fc3_b.reshape(E, 1, D))

    a, b = _make_combine_sc()(ys, dst0.reshape(N), dst1.reshape(N))
    out = _wadd_call(a, b, w0, w1)
    return out, aux.reshape(())


# speedup vs baseline: 1.0099x; 1.0027x over previous
import functools

import jax
import jax.numpy as jnp
from jax import lax
from jax.experimental import pallas as pl
from jax.experimental.pallas import tpu as pltpu
from jax.experimental.pallas import tpu_sc as plsc

N = 2048
D = 768
G = 16
EPG = 4
E = G * EPG
NP = 2 * N
B = 128
NB = 96
LP = NB * B
NC, NS = 2, 16
NW = NC * NS
ALPHA = 0.01


def _routing_body(x_ref, mw_ref, mb_ref, rw_ref, rb_ref,
                  dst0_ref, dst1_ref, w0_ref, w1_ref, be_ref, bv_ref,
                  nbu_ref, aux_ref):
    x = x_ref[...]
    ml = jnp.dot(x, mw_ref[...], preferred_element_type=jnp.float32) + mb_ref[...]
    rl = jnp.dot(x, rw_ref[...], preferred_element_type=jnp.float32) + rb_ref[...]

    gi = lax.broadcasted_iota(jnp.int32, (N, G), 1)
    mv = jnp.max(ml, axis=1, keepdims=True)
    tg = jnp.min(jnp.where(ml == mv, gi, G), axis=1, keepdims=True)

    me = jnp.exp(ml - mv)
    mp = me / jnp.sum(me, axis=1, keepdims=True)
    f = jnp.sum(mp, axis=0, keepdims=True) * (1.0 / N)
    aux = ALPHA * G * jnp.sum(f * f, axis=1, keepdims=True)

    gm = (tg == gi).astype(jnp.float32)
    counts = jnp.sum(gm, axis=0, keepdims=True)

    rlj = [rl[:, G * j:G * (j + 1)] for j in range(EPG)]
    mj = jnp.maximum(jnp.maximum(rlj[0], rlj[1]), jnp.maximum(rlj[2], rlj[3]))
    ej = [jnp.exp(t - mj) for t in rlj]
    se = ej[0] + ej[1] + ej[2] + ej[3]
    lbsum = jnp.zeros((1, G), jnp.float32)
    glj = []
    for j in range(EPG):
        pj = ej[j] / se
        fgj = jnp.sum(pj * gm, axis=0, keepdims=True) / jnp.maximum(counts, 1.0)
        lbsum = lbsum + fgj * fgj
        glj.append(jnp.sum(rlj[j] * gm, axis=1, keepdims=True))
    lb = ALPHA * EPG * lbsum
    aux = aux + jnp.sum(jnp.where(counts > 0.0, lb, 0.0), axis=1, keepdims=True)

    v0 = jnp.maximum(jnp.maximum(glj[0], glj[1]), jnp.maximum(glj[2], glj[3]))
    e0 = jnp.where(glj[0] == v0, 0,
         jnp.where(glj[1] == v0, 1,
         jnp.where(glj[2] == v0, 2, 3))).astype(jnp.int32)
    neg = jnp.float32(-1e30)
    gmj = [jnp.where(e0 == j, neg, glj[j]) for j in range(EPG)]
    v1 = jnp.maximum(jnp.maximum(gmj[0], gmj[1]), jnp.maximum(gmj[2], gmj[3]))
    e1 = jnp.where(gmj[0] == v1, 0,
         jnp.where(gmj[1] == v1, 1,
         jnp.where(gmj[2] == v1, 2, 3))).astype(jnp.int32)
    t = jnp.exp(v1 - v0)
    w0 = 1.0 / (1.0 + t)
    w1 = t / (1.0 + t)
    w0_ref[...] = w0
    w1_ref[...] = w1
    aux_ref[...] = aux

    hi = jax.lax.Precision.HIGHEST
    eid0 = tg * EPG + e0
    eid1 = tg * EPG + e1
    eI = lax.broadcasted_iota(jnp.int32, (N, E), 1)
    oh0 = (eid0 == eI).astype(jnp.float32)
    oh1 = (eid1 == eI).astype(jnp.float32)
    ohs = oh0 + oh1
    countf = jnp.sum(ohs, axis=0, keepdims=True)
    pcf = (((countf.astype(jnp.int32) + B - 1) // B) * B).astype(jnp.float32)
    r64 = lax.broadcasted_iota(jnp.int32, (E, E), 0)
    c64 = lax.broadcasted_iota(jnp.int32, (E, E), 1)
    tl = (r64 < c64).astype(jnp.float32)
    pstart = jnp.dot(pcf, tl, preferred_element_type=jnp.float32, precision=hi)

    cb = 128
    rcb = lax.broadcasted_iota(jnp.int32, (cb, cb), 0)
    ccb = lax.broadcasted_iota(jnp.int32, (cb, cb), 1)
    lex = (rcb > ccb).astype(jnp.float32)
    runtot = jnp.zeros((1, E), jnp.float32)
    d0s, d1s = [], []
    for k in range(N // cb):
        sl = slice(cb * k, cb * (k + 1))
        ex = jnp.dot(lex, ohs[sl], preferred_element_type=jnp.float32,
                     precision=hi) + runtot + pstart
        d0s.append(jnp.sum(ex * oh0[sl], axis=1, keepdims=True))
        d1s.append(jnp.sum(ex * oh1[sl], axis=1, keepdims=True))
        runtot = runtot + jnp.sum(ohs[sl], axis=0, keepdims=True)
    dst0_ref[...] = jnp.concatenate(d0s, axis=0).astype(jnp.int32)
    dst1_ref[...] = jnp.concatenate(d1s, axis=0).astype(jnp.int32)

    bs = (lax.broadcasted_iota(jnp.int32, (NB, 1), 0) * B).astype(jnp.float32)
    cmp = (pstart <= bs).astype(jnp.float32)
    be = jnp.clip(jnp.sum(cmp, axis=1, keepdims=True) - 1.0, 0.0, E - 1.0)
    total = jnp.sum(pcf, axis=1, keepdims=True)
    used = (bs < total).astype(jnp.float32)
    lastm = (bs + B >= total).astype(jnp.float32) * used
    lastbe = jnp.sum(be * lastm, axis=0, keepdims=True)
    be_ref[...] = (be * used + lastbe * (1.0 - used)).astype(jnp.int32)
    bv_ref[...] = used.astype(jnp.int32)
    nbu_ref[...] = (total * (1.0 / B)).astype(jnp.int32)


_routing_call = pl.pallas_call(
    _routing_body,
    out_shape=(
        jax.ShapeDtypeStruct((N, 1), jnp.int32),
        jax.ShapeDtypeStruct((N, 1), jnp.int32),
        jax.ShapeDtypeStruct((N, 1), jnp.float32),
        jax.ShapeDtypeStruct((N, 1), jnp.float32),
        jax.ShapeDtypeStruct((NB, 1), jnp.int32),
        jax.ShapeDtypeStruct((NB, 1), jnp.int32),
        jax.ShapeDtypeStruct((1, 1), jnp.int32),
        jax.ShapeDtypeStruct((1, 1), jnp.float32),
    ),
)


def _dispatch_body(x_hbm, dst0_hbm, dst1_hbm, out_hbm, d0_v, d1_v, rows_v, sem):
    nt = N // NW
    wid = lax.axis_index("s") * NC + lax.axis_index("c")
    base = wid * nt
    pltpu.sync_copy(dst0_hbm.at[pl.ds(base, nt)], d0_v)
    pltpu.sync_copy(dst1_hbm.at[pl.ds(base, nt)], d1_v)
    pltpu.sync_copy(x_hbm.at[pl.ds(base, nt)], rows_v)
    c0 = pltpu.async_copy(rows_v, out_hbm.at[d0_v], sem)
    c1 = pltpu.async_copy(rows_v, out_hbm.at[d1_v], sem)
    c0.wait()
    c1.wait()


def _make_dispatch_sc():
    return pl.kernel(
        _dispatch_body,
        out_type=jax.ShapeDtypeStruct((LP, D), jnp.float32),
        mesh=plsc.VectorSubcoreMesh(core_axis_name="c", subcore_axis_name="s",
                                    num_cores=NC, num_subcores=NS),
        scratch_types=[
            pltpu.VMEM((N // NW,), jnp.int32),
            pltpu.VMEM((N // NW,), jnp.int32),
            pltpu.VMEM((N // NW, D), jnp.float32),
            pltpu.SemaphoreType.DMA,
        ],
    )


def _mlp_body(be_ref, bv_ref, nbu_ref, xs_ref, w1_ref, b1_ref, w2_ref, b2_ref,
              w3_ref, b3_ref, out_ref):
    @pl.when(bv_ref[pl.program_id(0)] == 1)
    def _():
        xb = xs_ref[...]
        h = jnp.dot(xb, w1_ref[0], preferred_element_type=jnp.float32) + b1_ref[0]
        h = jnp.maximum(h, 0.0)
        h = jnp.dot(h, w2_ref[0], preferred_element_type=jnp.float32) + b2_ref[0] + xb
        h = jnp.maximum(h, 0.0)
        out_ref[...] = jnp.dot(h, w3_ref[0], preferred_element_type=jnp.float32) + b3_ref[0]


def _expert_spec(shape):
    return pl.BlockSpec(shape, lambda b, be, bv, nbu: (be[b], 0, 0))


def _row_spec():
    return pl.BlockSpec((B, D), lambda b, be, bv, nbu: (jnp.minimum(b, nbu[0] - 1), 0))


_mlp_call = pl.pallas_call(
    _mlp_body,
    grid_spec=pltpu.PrefetchScalarGridSpec(
        num_scalar_prefetch=3,
        grid=(NB,),
        in_specs=[
            _row_spec(),
            _expert_spec((1, D, D)),
            _expert_spec((1, 1, D)),
            _expert_spec((1, D, D)),
            _expert_spec((1, 1, D)),
            _expert_spec((1, D, D)),
            _expert_spec((1, 1, D)),
        ],
        out_specs=_row_spec(),
    ),
    out_shape=jax.ShapeDtypeStruct((LP, D), jnp.float32),
)


def _combine_body(ys_hbm, p0_hbm, p1_hbm, a_hbm, b_hbm,
                  i0_v, i1_v, a_v, b_v, sem):
    nt = N // NW
    wid = lax.axis_index("s") * NC + lax.axis_index("c")
    base = wid * nt
    pltpu.sync_copy(p0_hbm.at[pl.ds(base, nt)], i0_v)
    pltpu.sync_copy(p1_hbm.at[pl.ds(base, nt)], i1_v)
    c0 = pltpu.async_copy(ys_hbm.at[i0_v], a_v, sem)
    c1 = pltpu.async_copy(ys_hbm.at[i1_v], b_v, sem)
    c0.wait()
    c1.wait()
    pltpu.sync_copy(a_v, a_hbm.at[pl.ds(base, nt)])
    pltpu.sync_copy(b_v, b_hbm.at[pl.ds(base, nt)])


def _make_combine_sc():
    return pl.kernel(
        _combine_body,
        out_type=(jax.ShapeDtypeStruct((N, D), jnp.float32),
                  jax.ShapeDtypeStruct((N, D), jnp.float32)),
        mesh=plsc.VectorSubcoreMesh(core_axis_name="c", subcore_axis_name="s",
                                    num_cores=NC, num_subcores=NS),
        scratch_types=[
            pltpu.VMEM((N // NW,), jnp.int32),
            pltpu.VMEM((N // NW,), jnp.int32),
            pltpu.VMEM((N // NW, D), jnp.float32),
            pltpu.VMEM((N // NW, D), jnp.float32),
            pltpu.SemaphoreType.DMA,
        ],
    )


def _wadd_body(a_ref, b_ref, w0_ref, w1_ref, out_ref):
    out_ref[...] = a_ref[...] * w0_ref[...] + b_ref[...] * w1_ref[...]


_WB = 256
_wadd_call = pl.pallas_call(
    _wadd_body,
    grid=(N // _WB,),
    in_specs=[
        pl.BlockSpec((_WB, D), lambda i: (i, 0)),
        pl.BlockSpec((_WB, D), lambda i: (i, 0)),
        pl.BlockSpec((_WB, 1), lambda i: (i, 0)),
        pl.BlockSpec((_WB, 1), lambda i: (i, 0)),
    ],
    out_specs=pl.BlockSpec((_WB, D), lambda i: (i, 0)),
    out_shape=jax.ShapeDtypeStruct((N, D), jnp.float32),
)


def kernel(x, meta_W, meta_b, router_W, router_b,
           fc1_W, fc1_b, fc2_W, fc2_b, fc3_W, fc3_b):
    rwf = jnp.transpose(router_W, (2, 0, 1)).reshape(E, D).T
    rbf = jnp.transpose(router_b, (1, 0)).reshape(1, E)
    mb = meta_b.reshape(1, G)

    dst0, dst1, w0, w1, be, bvalid, nbu, aux = _routing_call(x, meta_W, mb, rwf, rbf)

    xs = _make_dispatch_sc()(x, dst0.reshape(N), dst1.reshape(N))

    ys = _mlp_call(be.reshape(NB), bvalid.reshape(NB), nbu.reshape(1), xs,
                   fc1_W.reshape(E, D, D), fc1_b.reshape(E, 1, D),
                   fc2_W.reshape(E, D, D), fc2_b.reshape(E, 1, D),
                   fc3_W.reshape(E, D, D), fc3_b.reshape(E, 1, D))

    a, b = _make_combine_sc()(ys, dst0.reshape(N), dst1.reshape(N))
    out = _wadd_call(a, b, w0, w1)
    return out, aux.reshape(())
